# Initial kernel scaffold; baseline (speedup 1.0000x reference)
import jax
import jax.numpy as jnp
from jax.experimental import pallas as pl


def kernel(x, x_sub, pos, pos_sub, W_sub, b_sub, gamma_sub, beta_sub, W, b, gamma, beta):
    raise NotImplementedError("write your pallas kernel here")


# speedup vs baseline: 11.5413x; 11.5413x over previous
import functools

import jax
import jax.numpy as jnp
from jax.experimental import pallas as pl
from jax.experimental.pallas import tpu as pltpu

_EPS_BN = 1e-5
_QB = 256


def _round_bf16(x):
    i = jax.lax.bitcast_convert_type(x, jnp.int32)
    r = (i + 0x7FFF + ((i >> 16) & 1)) & jnp.int32(-65536)
    return jax.lax.bitcast_convert_type(r, jnp.float32)


def _mlp_sub_kernel(x_ref, w_ref, p_ref, o_ref):
    h = jnp.dot(x_ref[...], w_ref[...], preferred_element_type=jnp.float32)
    h = h + p_ref[0, :][None, :]
    mu = jnp.mean(h, axis=0, keepdims=True)
    var = jnp.mean((h - mu) ** 2, axis=0, keepdims=True)
    h = (h - mu) / jnp.sqrt(var + _EPS_BN)
    h = h * p_ref[1, :][None, :] + p_ref[2, :][None, :]
    o_ref[...] = jnp.maximum(h, 0.0)


def _knn_g_kernel(x_ref, pos_ref, post_ref, hsub_ref, w_ref, p_ref,
                  interp_ref, g_ref, stats_ref, *, n_src):
    j = pl.program_id(0)

    g = jnp.dot(x_ref[...], w_ref[...], preferred_element_type=jnp.float32)
    g = g + p_ref[0, :][None, :]
    g_ref[...] = g

    psum = jnp.sum(g, axis=0)
    psumsq = jnp.sum(g * g, axis=0)
    zeros = jnp.zeros_like(psum)
    blk = jnp.stack([psum, psumsq] + [zeros] * 6)

    @pl.when(j == 0)
    def _():
        stats_ref[...] = blk

    @pl.when(j != 0)
    def _():
        stats_ref[...] += blk

    q = pos_ref[...]
    s = post_ref[...]
    qq = jnp.sum(q * q, axis=1, keepdims=True)
    ss = jnp.sum(s * s, axis=0, keepdims=True)
    qr = _round_bf16(q)
    sr = _round_bf16(s)
    cross = (qr[:, 0:1] * sr[0, :][None, :]
             + qr[:, 1:2] * sr[1, :][None, :]
             + qr[:, 2:3] * sr[2, :][None, :])
    d2 = qq - 2.0 * cross + ss

    iota = jax.lax.broadcasted_iota(jnp.int32, d2.shape, 1)
    wsel = jnp.zeros_like(d2)
    den = jnp.zeros((d2.shape[0], 1), dtype=jnp.float32)
    d2m = d2
    for _k in range(3):
        m = jnp.min(d2m, axis=1, keepdims=True)
        cand = jnp.where(d2m == m, iota, n_src)
        amin = jnp.min(cand, axis=1, keepdims=True)
        onehot = iota == amin
        wk = 1.0 / jnp.maximum(m, 1e-16)
        wsel = jnp.where(onehot, wk, wsel)
        den = den + wk
        d2m = jnp.where(onehot, jnp.float32(jnp.inf), d2m)

    num = jnp.dot(wsel, hsub_ref[...], preferred_element_type=jnp.float32)
    interp_ref[...] = num / den


def _finish_kernel(g_ref, interp_ref, stats_ref, p_ref, o_ref, *, n_rows):
    mu = stats_ref[0, :] * (1.0 / n_rows)
    var = stats_ref[1, :] * (1.0 / n_rows) - mu * mu
    h = (g_ref[...] - mu[None, :]) / jnp.sqrt(var + _EPS_BN)[None, :]
    h = h * p_ref[1, :][None, :] + p_ref[2, :][None, :]
    o_ref[...] = jnp.maximum(h, 0.0) + interp_ref[...]


def kernel(x, x_sub, pos, pos_sub, W_sub, b_sub, gamma_sub, beta_sub, W, b, gamma, beta):
    n, cout = x.shape
    n_sub = x_sub.shape[0]

    p_sub = jnp.stack([b_sub, gamma_sub, beta_sub] + [jnp.zeros_like(b_sub)] * 5)
    p_main = jnp.stack([b, gamma, beta] + [jnp.zeros_like(b)] * 5)

    h_sub = pl.pallas_call(
        _mlp_sub_kernel,
        out_shape=jax.ShapeDtypeStruct((n_sub, cout), jnp.float32),
    )(x_sub, W_sub, p_sub)

    post = jnp.zeros((8, n_sub), jnp.float32).at[0:3, :].set(pos_sub.T)

    nb = n // _QB
    x_interp, g, stats = pl.pallas_call(
        functools.partial(_knn_g_kernel, n_src=n_sub),
        grid=(nb,),
        in_specs=[
            pl.BlockSpec((_QB, cout), lambda j: (j, 0)),
            pl.BlockSpec((_QB, 3), lambda j: (j, 0)),
            pl.BlockSpec((8, n_sub), lambda j: (0, 0)),
            pl.BlockSpec((n_sub, cout), lambda j: (0, 0)),
            pl.BlockSpec((cout, cout), lambda j: (0, 0)),
            pl.BlockSpec((8, cout), lambda j: (0, 0)),
        ],
        out_specs=[
            pl.BlockSpec((_QB, cout), lambda j: (j, 0)),
            pl.BlockSpec((_QB, cout), lambda j: (j, 0)),
            pl.BlockSpec((8, cout), lambda j: (0, 0)),
        ],
        out_shape=[
            jax.ShapeDtypeStruct((n, cout), jnp.float32),
            jax.ShapeDtypeStruct((n, cout), jnp.float32),
            jax.ShapeDtypeStruct((8, cout), jnp.float32),
        ],
    )(x, pos, post, h_sub, W, p_main)

    out = pl.pallas_call(
        functools.partial(_finish_kernel, n_rows=n),
        grid=(nb,),
        in_specs=[
            pl.BlockSpec((_QB, cout), lambda j: (j, 0)),
            pl.BlockSpec((_QB, cout), lambda j: (j, 0)),
            pl.BlockSpec((8, cout), lambda j: (0, 0)),
            pl.BlockSpec((8, cout), lambda j: (0, 0)),
        ],
        out_specs=pl.BlockSpec((_QB, cout), lambda j: (j, 0)),
        out_shape=jax.ShapeDtypeStruct((n, cout), jnp.float32),
    )(g, x_interp, stats, p_main)
    return out
